# Initial kernel scaffold; baseline (speedup 1.0000x reference)
import jax
import jax.numpy as jnp
from jax.experimental import pallas as pl


def kernel(atom_batch, x, W, b):
    raise NotImplementedError("write your pallas kernel here")


# speedup vs baseline: 1.9742x; 1.9742x over previous
import functools

import jax
import jax.numpy as jnp
from jax import lax
from jax.experimental import pallas as pl
from jax.experimental.pallas import tpu as pltpu
from jax.experimental.pallas import tpu_sc as plsc

N = 320000
D = 128
NSEG = 10000

NC = 2
NS = 16
L = 16

NSEG_PAD = 10240
SLICE = NSEG_PAD // NS
CHUNK = N // (NC * NS)
MV_B = 2560


def _matvec_body(x_ref, w_ref, b_ref, o_ref):
    xb = x_ref[0]
    s = jnp.sum(xb * w_ref[...], axis=1)
    o_ref[...] = (s + b_ref[0, 0]).reshape(1, 1, MV_B)


def _matvec(x, w_row, b11):
    grid = N // MV_B
    x3 = x.reshape(grid, MV_B, D)
    out = pl.pallas_call(
        _matvec_body,
        grid=(grid,),
        in_specs=[
            pl.BlockSpec((1, MV_B, D), lambda i: (i, 0, 0)),
            pl.BlockSpec((1, D), lambda i: (0, 0)),
            pl.BlockSpec((1, 1), lambda i: (0, 0), memory_space=pltpu.SMEM),
        ],
        out_specs=pl.BlockSpec((1, 1, MV_B), lambda i: (i, 0, 0)),
        out_shape=jax.ShapeDtypeStruct((grid, 1, MV_B), jnp.float32),
    )(x3, w_row, b11)
    return out.reshape(N)


def _segsum_body(batch_hbm, y_hbm, out_hbm, idx_v, y_v, acc, stage, acc2):
    c = lax.axis_index("c")
    s = lax.axis_index("s")
    wid = s * NC + c
    base = wid * CHUNK

    pltpu.sync_copy(batch_hbm.at[pl.ds(base, CHUNK)], idx_v)
    pltpu.sync_copy(y_hbm.at[pl.ds(base, CHUNK)], y_v)

    zero = jnp.zeros((L,), jnp.float32)

    def zbody(i, _):
        acc[pl.ds(i * L, L)] = zero
        return 0

    lax.fori_loop(0, NSEG_PAD // L, zbody, 0)

    def sbody(i, _):
        ids = idx_v[pl.ds(i * L, L)]
        vals = y_v[pl.ds(i * L, L)]
        plsc.addupdate_scatter(acc, [ids], vals)
        return 0

    lax.fori_loop(0, CHUNK // L, sbody, 0)

    pltpu.sync_copy(acc, stage.at[s])
    plsc.subcore_barrier()

    def zbody2(i, _):
        acc2[pl.ds(i * L, L)] = zero
        return 0

    lax.fori_loop(0, SLICE // L, zbody2, 0)

    def rbody(k, _):
        pltpu.sync_copy(stage.at[k, pl.ds(s * SLICE, SLICE)], y_v.at[pl.ds(0, SLICE)])

        def abody(j, _):
            sl = pl.ds(j * L, L)
            acc2[sl] = acc2[sl] + y_v[sl]
            return 0

        lax.fori_loop(0, SLICE // L, abody, 0)
        return 0

    lax.fori_loop(0, NS, rbody, 0)

    pltpu.sync_copy(acc2, out_hbm.at[pl.ds(c * NSEG_PAD + s * SLICE, SLICE)])


@functools.cache
def _make_segsum():
    return pl.kernel(
        _segsum_body,
        out_type=jax.ShapeDtypeStruct((NC * NSEG_PAD,), jnp.float32),
        mesh=plsc.VectorSubcoreMesh(core_axis_name="c", subcore_axis_name="s"),
        compiler_params=pltpu.CompilerParams(needs_layout_passes=False),
        scratch_types=[
            pltpu.VMEM((CHUNK,), jnp.int32),
            pltpu.VMEM((CHUNK,), jnp.float32),
            pltpu.VMEM((NSEG_PAD,), jnp.float32),
            pltpu.VMEM_SHARED((NS, NSEG_PAD), jnp.float32),
            pltpu.VMEM((SLICE,), jnp.float32),
        ],
    )


def kernel(atom_batch, x, W, b):
    ids = atom_batch.astype(jnp.int32)
    w_row = W.reshape(1, D).astype(jnp.float32)
    b11 = b.reshape(1, 1).astype(jnp.float32)
    y = _matvec(x, w_row, b11)
    partials = _make_segsum()(ids, y)
    per_core = partials.reshape(NC, NSEG_PAD)
    return (per_core[0] + per_core[1])[:NSEG]


# speedup vs baseline: 2.7399x; 1.3878x over previous
import functools

import jax
import jax.numpy as jnp
from jax import lax
from jax.experimental import pallas as pl
from jax.experimental.pallas import tpu as pltpu
from jax.experimental.pallas import tpu_sc as plsc

N = 320000
D = 128
NSEG = 10000

NC = 2
NS = 16
L = 16

NSEG_PAD = 10240
SLICE = NSEG_PAD // NS
CHUNK = N // (NC * NS)
MV_B = 2560


def _matvec_body(x_ref, w_ref, b_ref, o_ref):
    xb = x_ref[0]
    s = jax.lax.dot_general(
        w_ref[...], xb, (((1,), (1,)), ((), ())),
        preferred_element_type=jnp.float32,
    )
    o_ref[...] = (s + b_ref[0, 0]).reshape(1, 1, MV_B)


def _matvec(x, w_row, b11):
    grid = N // MV_B
    x3 = x.reshape(grid, MV_B, D)
    out = pl.pallas_call(
        _matvec_body,
        grid=(grid,),
        in_specs=[
            pl.BlockSpec((1, MV_B, D), lambda i: (i, 0, 0)),
            pl.BlockSpec((1, D), lambda i: (0, 0)),
            pl.BlockSpec((1, 1), lambda i: (0, 0), memory_space=pltpu.SMEM),
        ],
        out_specs=pl.BlockSpec((1, 1, MV_B), lambda i: (i, 0, 0)),
        out_shape=jax.ShapeDtypeStruct((grid, 1, MV_B), jnp.float32),
    )(x3, w_row, b11)
    return out.reshape(N)


def _segsum_body(batch_hbm, y_hbm, out_hbm, idx_v, y_v, acc, stage, acc2):
    c = lax.axis_index("c")
    s = lax.axis_index("s")
    wid = s * NC + c
    base = wid * CHUNK

    pltpu.sync_copy(batch_hbm.at[pl.ds(base, CHUNK)], idx_v)
    pltpu.sync_copy(y_hbm.at[pl.ds(base, CHUNK)], y_v)

    zero = jnp.zeros((L,), jnp.float32)

    def zbody(i, _):
        acc[pl.ds(i * L, L)] = zero
        return 0

    lax.fori_loop(0, NSEG_PAD // L, zbody, 0)

    def sbody(i, _):
        ids = idx_v[pl.ds(i * L, L)]
        vals = y_v[pl.ds(i * L, L)]
        plsc.addupdate_scatter(acc, [ids], vals)
        return 0

    lax.fori_loop(0, CHUNK // L, sbody, 0)

    pltpu.sync_copy(acc, stage.at[s])
    plsc.subcore_barrier()

    def zbody2(i, _):
        acc2[pl.ds(i * L, L)] = zero
        return 0

    lax.fori_loop(0, SLICE // L, zbody2, 0)

    def rbody(k, _):
        pltpu.sync_copy(stage.at[k, pl.ds(s * SLICE, SLICE)], y_v.at[pl.ds(0, SLICE)])

        def abody(j, _):
            sl = pl.ds(j * L, L)
            acc2[sl] = acc2[sl] + y_v[sl]
            return 0

        lax.fori_loop(0, SLICE // L, abody, 0)
        return 0

    lax.fori_loop(0, NS, rbody, 0)

    pltpu.sync_copy(acc2, out_hbm.at[pl.ds(c * NSEG_PAD + s * SLICE, SLICE)])


@functools.cache
def _make_segsum():
    return pl.kernel(
        _segsum_body,
        out_type=jax.ShapeDtypeStruct((NC * NSEG_PAD,), jnp.float32),
        mesh=plsc.VectorSubcoreMesh(core_axis_name="c", subcore_axis_name="s"),
        compiler_params=pltpu.CompilerParams(needs_layout_passes=False),
        scratch_types=[
            pltpu.VMEM((CHUNK,), jnp.int32),
            pltpu.VMEM((CHUNK,), jnp.float32),
            pltpu.VMEM((NSEG_PAD,), jnp.float32),
            pltpu.VMEM_SHARED((NS, NSEG_PAD), jnp.float32),
            pltpu.VMEM((SLICE,), jnp.float32),
        ],
    )


def kernel(atom_batch, x, W, b):
    ids = atom_batch.astype(jnp.int32)
    w_row = W.reshape(1, D).astype(jnp.float32)
    b11 = b.reshape(1, 1).astype(jnp.float32)
    y = _matvec(x, w_row, b11)
    partials = _make_segsum()(ids, y)
    per_core = partials.reshape(NC, NSEG_PAD)
    return (per_core[0] + per_core[1])[:NSEG]


# speedup vs baseline: 3.6690x; 1.3391x over previous
import functools

import jax
import jax.numpy as jnp
from jax import lax
from jax.experimental import pallas as pl
from jax.experimental.pallas import tpu as pltpu
from jax.experimental.pallas import tpu_sc as plsc

N = 320000
D = 128
NSEG = 10000

NC = 2
NS = 16
L = 16

NSEG_PAD = 10240
SLICE = NSEG_PAD // NS
CHUNK = N // (NC * NS)
MV_B = 6400


def _matvec_body(x_ref, w_ref, b_ref, o_ref):
    xb = x_ref[0]
    s = jax.lax.dot_general(
        w_ref[...], xb, (((1,), (1,)), ((), ())),
        preferred_element_type=jnp.float32,
    )
    o_ref[...] = (s + b_ref[0, 0]).reshape(1, 1, MV_B)


def _matvec(x, w_row, b11):
    grid = N // MV_B
    x3 = x.reshape(grid, MV_B, D)
    out = pl.pallas_call(
        _matvec_body,
        grid=(grid,),
        in_specs=[
            pl.BlockSpec((1, MV_B, D), lambda i: (i, 0, 0)),
            pl.BlockSpec((1, D), lambda i: (0, 0)),
            pl.BlockSpec((1, 1), lambda i: (0, 0), memory_space=pltpu.SMEM),
        ],
        out_specs=pl.BlockSpec((1, 1, MV_B), lambda i: (i, 0, 0)),
        out_shape=jax.ShapeDtypeStruct((grid, 1, MV_B), jnp.float32),
    )(x3, w_row, b11)
    return out.reshape(N)


def _segsum_body(batch_hbm, y_hbm, out_hbm, idx_v, y_v, acc, stage, acc2):
    c = lax.axis_index("c")
    s = lax.axis_index("s")
    wid = s * NC + c
    base = wid * CHUNK

    pltpu.sync_copy(batch_hbm.at[pl.ds(base, CHUNK)], idx_v)
    pltpu.sync_copy(y_hbm.at[pl.ds(base, CHUNK)], y_v)

    zero = jnp.zeros((L,), jnp.float32)

    def zbody(i, _):
        acc[pl.ds(i * L, L)] = zero
        return 0

    lax.fori_loop(0, NSEG_PAD // L, zbody, 0)

    def sbody(i, _):
        ids = idx_v[pl.ds(i * L, L)]
        vals = y_v[pl.ds(i * L, L)]
        plsc.addupdate_scatter(acc, [ids], vals)
        return 0

    lax.fori_loop(0, CHUNK // L, sbody, 0)

    pltpu.sync_copy(acc, stage.at[s])
    plsc.subcore_barrier()

    def zbody2(i, _):
        acc2[pl.ds(i * L, L)] = zero
        return 0

    lax.fori_loop(0, SLICE // L, zbody2, 0)

    def rbody(k, _):
        pltpu.sync_copy(stage.at[k, pl.ds(s * SLICE, SLICE)], y_v.at[pl.ds(0, SLICE)])

        def abody(j, _):
            sl = pl.ds(j * L, L)
            acc2[sl] = acc2[sl] + y_v[sl]
            return 0

        lax.fori_loop(0, SLICE // L, abody, 0)
        return 0

    lax.fori_loop(0, NS, rbody, 0)

    pltpu.sync_copy(acc2, out_hbm.at[pl.ds(c * NSEG_PAD + s * SLICE, SLICE)])


@functools.cache
def _make_segsum():
    return pl.kernel(
        _segsum_body,
        out_type=jax.ShapeDtypeStruct((NC * NSEG_PAD,), jnp.float32),
        mesh=plsc.VectorSubcoreMesh(core_axis_name="c", subcore_axis_name="s"),
        compiler_params=pltpu.CompilerParams(needs_layout_passes=False),
        scratch_types=[
            pltpu.VMEM((CHUNK,), jnp.int32),
            pltpu.VMEM((CHUNK,), jnp.float32),
            pltpu.VMEM((NSEG_PAD,), jnp.float32),
            pltpu.VMEM_SHARED((NS, NSEG_PAD), jnp.float32),
            pltpu.VMEM((SLICE,), jnp.float32),
        ],
    )


def kernel(atom_batch, x, W, b):
    ids = atom_batch.astype(jnp.int32)
    w_row = W.reshape(1, D).astype(jnp.float32)
    b11 = b.reshape(1, 1).astype(jnp.float32)
    y = _matvec(x, w_row, b11)
    partials = _make_segsum()(ids, y)
    per_core = partials.reshape(NC, NSEG_PAD)
    return (per_core[0] + per_core[1])[:NSEG]


# speedup vs baseline: 4.1989x; 1.1444x over previous
import functools

import jax
import jax.numpy as jnp
from jax import lax
from jax.experimental import pallas as pl
from jax.experimental.pallas import tpu as pltpu
from jax.experimental.pallas import tpu_sc as plsc

N = 320000
D = 128
NSEG = 10000

NC = 2
NS = 16
L = 16

NSEG_PAD = 10240
SLICE = NSEG_PAD // NS
CHUNK = N // (NC * NS)
MV_B = 12800


def _matvec_body(x_ref, w_ref, b_ref, o_ref):
    xb = x_ref[0]
    s = jax.lax.dot_general(
        w_ref[...], xb, (((1,), (1,)), ((), ())),
        preferred_element_type=jnp.float32,
    )
    o_ref[...] = (s + b_ref[0, 0]).reshape(1, 1, MV_B)


def _matvec(x, w_row, b11):
    grid = N // MV_B
    x3 = x.reshape(grid, MV_B, D)
    out = pl.pallas_call(
        _matvec_body,
        grid=(grid,),
        in_specs=[
            pl.BlockSpec((1, MV_B, D), lambda i: (i, 0, 0)),
            pl.BlockSpec((1, D), lambda i: (0, 0)),
            pl.BlockSpec((1, 1), lambda i: (0, 0), memory_space=pltpu.SMEM),
        ],
        out_specs=pl.BlockSpec((1, 1, MV_B), lambda i: (i, 0, 0)),
        out_shape=jax.ShapeDtypeStruct((grid, 1, MV_B), jnp.float32),
    )(x3, w_row, b11)
    return out.reshape(N)


def _segsum_body(batch_hbm, y_hbm, out_hbm, idx_v, y_v, acc, stage, acc2):
    c = lax.axis_index("c")
    s = lax.axis_index("s")
    wid = s * NC + c
    base = wid * CHUNK

    pltpu.sync_copy(batch_hbm.at[pl.ds(base, CHUNK)], idx_v)
    pltpu.sync_copy(y_hbm.at[pl.ds(base, CHUNK)], y_v)

    zero = jnp.zeros((L,), jnp.float32)

    def zbody(i, _):
        acc[pl.ds(i * L, L)] = zero
        return 0

    lax.fori_loop(0, NSEG_PAD // L, zbody, 0)

    def sbody(i, _):
        ids = idx_v[pl.ds(i * L, L)]
        vals = y_v[pl.ds(i * L, L)]
        plsc.addupdate_scatter(acc, [ids], vals)
        return 0

    lax.fori_loop(0, CHUNK // L, sbody, 0)

    pltpu.sync_copy(acc, stage.at[s])
    plsc.subcore_barrier()

    def zbody2(i, _):
        acc2[pl.ds(i * L, L)] = zero
        return 0

    lax.fori_loop(0, SLICE // L, zbody2, 0)

    def rbody(k, _):
        pltpu.sync_copy(stage.at[k, pl.ds(s * SLICE, SLICE)], y_v.at[pl.ds(0, SLICE)])

        def abody(j, _):
            sl = pl.ds(j * L, L)
            acc2[sl] = acc2[sl] + y_v[sl]
            return 0

        lax.fori_loop(0, SLICE // L, abody, 0)
        return 0

    lax.fori_loop(0, NS, rbody, 0)

    pltpu.sync_copy(acc2, out_hbm.at[pl.ds(c * NSEG_PAD + s * SLICE, SLICE)])


@functools.cache
def _make_segsum():
    return pl.kernel(
        _segsum_body,
        out_type=jax.ShapeDtypeStruct((NC * NSEG_PAD,), jnp.float32),
        mesh=plsc.VectorSubcoreMesh(core_axis_name="c", subcore_axis_name="s"),
        compiler_params=pltpu.CompilerParams(needs_layout_passes=False),
        scratch_types=[
            pltpu.VMEM((CHUNK,), jnp.int32),
            pltpu.VMEM((CHUNK,), jnp.float32),
            pltpu.VMEM((NSEG_PAD,), jnp.float32),
            pltpu.VMEM_SHARED((NS, NSEG_PAD), jnp.float32),
            pltpu.VMEM((SLICE,), jnp.float32),
        ],
    )


def kernel(atom_batch, x, W, b):
    ids = atom_batch.astype(jnp.int32)
    w_row = W.reshape(1, D).astype(jnp.float32)
    b11 = b.reshape(1, 1).astype(jnp.float32)
    y = _matvec(x, w_row, b11)
    partials = _make_segsum()(ids, y)
    per_core = partials.reshape(NC, NSEG_PAD)
    return (per_core[0] + per_core[1])[:NSEG]
